# Initial kernel scaffold; baseline (speedup 1.0000x reference)
import jax
import jax.numpy as jnp
from jax.experimental import pallas as pl


def kernel(x, edge_index, W1, a_src1, a_dst1, b1, W2, a_src2, a_dst2, b2):
    raise NotImplementedError("write your pallas kernel here")


# speedup vs baseline: 18.5125x; 18.5125x over previous
import functools

import jax
import jax.numpy as jnp
from jax import lax
from jax.experimental import pallas as pl
from jax.experimental.pallas import tpu as pltpu
from jax.experimental.pallas import tpu_sc as plsc

N = 10000
E = 320000
IN = 128
H1 = 4
C1 = 64
HC1 = H1 * C1
OUT = 64

G = 80
NTILE = 16
ROWS_PER_TILE = N // NTILE


_SC_PARAMS = pltpu.CompilerParams(
    use_tc_tiling_on_sc=False, needs_layout_passes=False)
_MESH = plsc.VectorSubcoreMesh(core_axis_name="c", subcore_axis_name="s")


def _att_kernel(P, EPT):
    NCH = EPT // G

    def body(src_r, dstf_r, as_r, ad_r, p_r,
             src_v, dst_f, as_v, ad_v, p_v):
        c = lax.axis_index("c")
        s = lax.axis_index("s")
        pltpu.sync_copy(src_r.at[c, s], src_v)
        pltpu.sync_copy(dstf_r.at[c, s], dst_f)
        pltpu.sync_copy(as_r.at[c], as_v)
        pltpu.sync_copy(ad_r.at[c], ad_v)

        def chunk(g, _):
            off = g * G
            for j in range(G // 16):
                sv = src_v[pl.ds(off + j * 16, 16)]
                dv = dst_f[pl.ds(off + j * 16, 16)]
                for h in range(P):
                    hoff = jnp.full((16,), h, jnp.int32)
                    a_s = plsc.load_gather(as_v, [sv * P + hoff])
                    a_d = plsc.load_gather(ad_v, [dv * P + hoff])
                    e = a_s + a_d
                    p = jnp.exp(jnp.maximum(e, 0.2 * e))
                    p_v[pl.ds(off * P + h * G + j * 16, 16)] = p
            return _

        lax.fori_loop(0, NCH, chunk, None)
        pltpu.sync_copy(p_v, p_r.at[c, s])

    return functools.partial(
        pl.kernel,
        mesh=_MESH,
        compiler_params=_SC_PARAMS,
        out_type=jax.ShapeDtypeStruct((2, NTILE, EPT * P), jnp.float32),
        scratch_types=[
            pltpu.VMEM((EPT,), jnp.int32),
            pltpu.VMEM((EPT,), jnp.int32),
            pltpu.VMEM((N * P,), jnp.float32),
            pltpu.VMEM((N * P,), jnp.float32),
            pltpu.VMEM((EPT * P,), jnp.float32),
        ],
    )(body)


def _msg_kernel(W, P, EPT):
    AW = W + 16
    NCH = EPT // G
    CPH = 64 // 16

    def body(rec_r, dst_r, h_r, out_r, acc, recv, dst_cv, rows, msg, zb, sem):
        c = lax.axis_index("c")
        s = lax.axis_index("s")

        zeros16 = jnp.zeros((16,), jnp.float32)
        for i in range(25):
            for j in range(AW // 16):
                zb[i, pl.ds(j * 16, 16)] = zeros16
        for k in range(ROWS_PER_TILE // 25):
            pltpu.sync_copy(zb, acc.at[pl.ds(s * ROWS_PER_TILE + k * 25, 25)])
        plsc.subcore_barrier()

        iota16 = lax.iota(jnp.int32, 16)

        def chunk(g, _):
            pltpu.sync_copy(rec_r.at[c, s, g], recv)
            pltpu.sync_copy(dst_r.at[c, s, g], dst_cv)
            cp = pltpu.async_copy(h_r.at[recv.at[pl.ds(0, G)]], rows, sem)
            cp.wait()

            def edge(i, _):
                pvec = zeros16
                for h in range(P):
                    pv = plsc.bitcast(plsc.load_gather(
                        recv, [jnp.full((16,), (1 + h) * G, jnp.int32) + i]),
                        jnp.float32)
                    pvec = pvec + jnp.where(iota16 == h, pv, 0.0)
                    for j in range(CPH):
                        col = h * 64 + j * 16
                        msg[i, pl.ds(col, 16)] = rows[i, pl.ds(col, 16)] * pv
                msg[i, pl.ds(W, 16)] = pvec
                return _

            lax.fori_loop(0, G, edge, None)
            pltpu.sync_copy(msg, acc.at[dst_cv], add=True)
            return _

        lax.fori_loop(0, NCH, chunk, None)
        plsc.subcore_barrier()
        pltpu.sync_copy(acc.at[pl.ds(s * ROWS_PER_TILE, ROWS_PER_TILE)],
                        out_r.at[c, pl.ds(s * ROWS_PER_TILE, ROWS_PER_TILE)])

    return functools.partial(
        pl.kernel,
        mesh=_MESH,
        compiler_params=_SC_PARAMS,
        out_type=jax.ShapeDtypeStruct((2, N, AW), jnp.float32),
        scratch_types=[
            pltpu.VMEM_SHARED((N, AW), jnp.float32),
            pltpu.VMEM(((1 + P) * G,), jnp.int32),
            pltpu.VMEM((G,), jnp.int32),
            pltpu.VMEM((G, W), jnp.float32),
            pltpu.VMEM((G, AW), jnp.float32),
            pltpu.VMEM((25, AW), jnp.float32),
            pltpu.SemaphoreType.DMA,
        ],
    )(body)


_att1 = _att_kernel(2, E // NTILE)
_att2 = _att_kernel(1, E // (2 * NTILE))
_msg1 = _msg_kernel(128, 2, E // NTILE)
_msg2 = _msg_kernel(64, 1, E // (2 * NTILE))


def _run_edges(att, msg, src_plain, src_off, dstf, as_t, ad_t, htab, P, EPT):
    NCH = EPT // G
    p = att(src_plain, dstf, as_t, ad_t)
    rec = jnp.concatenate(
        [src_off.reshape(2, NTILE, NCH, 1, G),
         lax.bitcast_convert_type(p, jnp.int32).reshape(2, NTILE, NCH, P, G)],
        axis=3).reshape(2, NTILE, NCH, (1 + P) * G)
    dst_c = dstf.reshape(2, NTILE, NCH, G)
    return msg(rec, dst_c, htab)


def _tc_pre(x_ref, w_ref, a_ref, h_ref, asad_ref):
    h = jnp.dot(x_ref[...], w_ref[...], preferred_element_type=jnp.float32)
    h_ref[...] = h
    asad_ref[...] = jnp.dot(h, a_ref[...], preferred_element_type=jnp.float32)


def _tc_mid(accA_ref, accB_ref, h_ref, asad_ref, w2_ref, a2_ref, b1_ref,
            h2_ref, asad2_ref):
    asad = asad_ref[...]
    p_self = jnp.exp(jax.nn.leaky_relu(asad[:, :H1] + asad[:, H1:], 0.2))
    h = h_ref[...]
    cols = []
    for hd in range(H1):
        acc = accA_ref[...] if hd < 2 else accB_ref[...]
        j = hd % 2
        ps = p_self[:, hd:hd + 1]
        num = acc[:, j * 64:(j + 1) * 64] + ps * h[:, hd * 64:(hd + 1) * 64]
        den = acc[:, 128 + j:129 + j] + ps + 1e-16
        cols.append(num / den)
    out1 = jnp.concatenate(cols, axis=1) + b1_ref[...]
    x2 = jnp.where(out1 > 0, out1, jnp.exp(jnp.minimum(out1, 0.0)) - 1.0)
    h2 = jnp.dot(x2, w2_ref[...], preferred_element_type=jnp.float32)
    h2_ref[...] = h2
    asad2_ref[...] = jnp.dot(h2, a2_ref[...], preferred_element_type=jnp.float32)


def _tc_fin(accA_ref, accB_ref, h2_ref, asad2_ref, b2_ref, out_ref):
    asad2 = asad2_ref[...]
    p_self = jnp.exp(jax.nn.leaky_relu(asad2[:, :1] + asad2[:, 1:], 0.2))
    num = accA_ref[:, :OUT] + accB_ref[:, :OUT] + p_self * h2_ref[...]
    den = accA_ref[:, OUT:OUT + 1] + accB_ref[:, OUT:OUT + 1] + p_self + 1e-16
    out_ref[...] = num / den + b2_ref[...]


_GRID = 10
_BR = N // _GRID


def _row_spec(w):
    return pl.BlockSpec((_BR, w), lambda i: (i, 0))


def _full_spec(r, c):
    return pl.BlockSpec((r, c), lambda i: (0, 0))


def kernel(x, edge_index, W1, a_src1, a_dst1, b1, W2, a_src2, a_dst2, b2):
    f32 = jnp.float32
    src = edge_index[0]
    dst = edge_index[1]

    A1 = jnp.concatenate(
        [jax.scipy.linalg.block_diag(*[a_src1[h][:, None] for h in range(H1)]),
         jax.scipy.linalg.block_diag(*[a_dst1[h][:, None] for h in range(H1)])],
        axis=1)
    A2 = jnp.concatenate([a_src2.T, a_dst2.T], axis=1)

    h1, asad1 = pl.pallas_call(
        _tc_pre,
        grid=(_GRID,),
        in_specs=[_row_spec(IN), _full_spec(IN, HC1), _full_spec(HC1, 2 * H1)],
        out_specs=[_row_spec(HC1), _row_spec(2 * H1)],
        out_shape=[jax.ShapeDtypeStruct((N, HC1), f32),
                   jax.ShapeDtypeStruct((N, 2 * H1), f32)],
    )(x, W1, A1)

    htab1 = jnp.swapaxes(h1.reshape(N, 2, 128), 0, 1).reshape(2 * N, 128)
    as1 = jnp.swapaxes(asad1[:, :H1].reshape(N, 2, 2), 0, 1).reshape(2, N * 2)
    ad1 = jnp.swapaxes(asad1[:, H1:].reshape(N, 2, 2), 0, 1).reshape(2, N * 2)
    srcr = src.reshape(NTILE, E // NTILE)
    src1p = jnp.stack([srcr, srcr])
    src1o = jnp.stack([srcr, srcr + N])
    dstfr = dst.reshape(NTILE, E // NTILE)
    dstf1 = jnp.stack([dstfr, dstfr])
    acc1 = _run_edges(_att1, _msg1, src1p, src1o, dstf1, as1, ad1, htab1,
                      2, E // NTILE)

    b1r = b1.reshape(1, HC1)
    h2, asad2 = pl.pallas_call(
        _tc_mid,
        grid=(_GRID,),
        in_specs=[_row_spec(144), _row_spec(144), _row_spec(HC1),
                  _row_spec(2 * H1), _full_spec(HC1, OUT), _full_spec(OUT, 2),
                  _full_spec(1, HC1)],
        out_specs=[_row_spec(OUT), _row_spec(2)],
        out_shape=[jax.ShapeDtypeStruct((N, OUT), f32),
                   jax.ShapeDtypeStruct((N, 2), f32)],
    )(acc1[0], acc1[1], h1, asad1, W2, A2, b1r)

    as2 = jnp.broadcast_to(asad2[:, 0], (2, N))
    ad2 = jnp.broadcast_to(asad2[:, 1], (2, N))
    src2 = src.reshape(2, NTILE, E // (2 * NTILE))
    dstf2 = dst.reshape(2, NTILE, E // (2 * NTILE))
    acc2 = _run_edges(_att2, _msg2, src2, src2, dstf2, as2, ad2, h2,
                      1, E // (2 * NTILE))

    b2r = b2.reshape(1, OUT)
    out = pl.pallas_call(
        _tc_fin,
        grid=(_GRID,),
        in_specs=[_row_spec(80), _row_spec(80), _row_spec(OUT),
                  _row_spec(2), _full_spec(1, OUT)],
        out_specs=_row_spec(OUT),
        out_shape=jax.ShapeDtypeStruct((N, OUT), f32),
    )(acc2[0], acc2[1], h2, asad2, b2r)
    return out
